# baseline (speedup 1.0000x reference)
import functools

import jax
import jax.numpy as jnp
from jax import lax
from jax.experimental import pallas as pl
from jax.experimental.pallas import tpu as pltpu
from jax.experimental.pallas import tpu_sc as plsc

_TN = 512
_KC = 4096


def _dot(a, b):
    return lax.dot_general(a, b, (((1,), (0,)), ((), ())),
                           preferred_element_type=jnp.float32)


def _enc_vq_body(x_ref, w1_ref, b1_ref, w2_ref, b2_ref, cbt_ref, cbtb_ref,
                 z_ref, idx_ref, bsq_ref):
    K = cbt_ref.shape[1]

    @pl.when(pl.program_id(0) == 0)
    def _():
        cbt = cbt_ref[...]
        bsq_ref[...] = jnp.sum(cbt * cbt, axis=0, keepdims=True)

    h = jnp.maximum(_dot(x_ref[...].astype(jnp.bfloat16), w1_ref[...])
                    + b1_ref[...], 0.0)
    z = _dot(h.astype(jnp.bfloat16), w2_ref[...]) + b2_ref[...]
    a = jnp.sum(z * z, axis=1, keepdims=True)
    zb2 = (z * -2.0).astype(jnp.bfloat16)
    tn = z.shape[0]
    kc = min(_KC, K)
    run_min = None
    for c in range(0, K, kc):
        m2 = _dot(zb2, cbtb_ref[:, c:c + kc])
        d = (a + bsq_ref[:, c:c + kc]) + m2
        cmin = jnp.min(d, axis=1, keepdims=True)
        iota = lax.broadcasted_iota(jnp.int32, (1, kc), 1) + c
        cidx = jnp.min(jnp.where(d == cmin, iota, K), axis=1, keepdims=True)
        if run_min is None:
            run_min, run_idx = cmin, cidx
        else:
            better = cmin < run_min
            run_min = jnp.where(better, cmin, run_min)
            run_idx = jnp.where(better, cidx, run_idx)
    z_ref[...] = z
    idx_ref[0, 0, :] = run_idx.reshape(tn)


def _dec_body(q_ref, z_ref, w1_ref, b1_ref, w2_ref, b2_ref,
              xr_ref, loss_ref):
    q = q_ref[...].astype(jnp.bfloat16).astype(jnp.float32)
    z = z_ref[...]
    diff = q - z

    @pl.when(pl.program_id(0) == 0)
    def _():
        loss_ref[...] = jnp.zeros_like(loss_ref)

    loss_ref[...] += jnp.sum(diff * diff)
    qst = z + diff
    hd = jnp.maximum(_dot(qst.astype(jnp.bfloat16), w1_ref[...]) + b1_ref[...],
                     0.0)
    logits = _dot(hd.astype(jnp.bfloat16), w2_ref[...]) + b2_ref[...]
    xr_ref[...] = 1.0 / (1.0 + jnp.exp(-logits))


def _sc_gather(table, idx):
    n, d = idx.shape[0], table.shape[1]
    nw = 32
    bpw = n // nw
    mesh = plsc.VectorSubcoreMesh(core_axis_name="c", subcore_axis_name="s")

    @functools.partial(
        pl.kernel,
        mesh=mesh,
        out_type=jax.ShapeDtypeStruct((n, d), jnp.float32),
        scratch_types=[
            pltpu.VMEM((bpw,), jnp.int32),
            pltpu.VMEM((bpw, d), jnp.float32),
            pltpu.SemaphoreType.DMA,
        ],
    )
    def gather(table_hbm, idx_hbm, out_hbm, idx_v, rows_v, sem):
        wid = lax.axis_index("s") * 2 + lax.axis_index("c")
        base = wid * bpw
        pltpu.sync_copy(idx_hbm.at[pl.ds(base, bpw)], idx_v)
        pltpu.async_copy(table_hbm.at[idx_v], rows_v, sem).wait()
        pltpu.sync_copy(rows_v, out_hbm.at[pl.ds(base, bpw)])

    return gather(table, idx)


def _enc_vq(xp, w1b, b1r, w2b, b2r, cbt, cbtb):
    n, D = xp.shape
    H = w1b.shape[1]
    L = w2b.shape[1]
    K = cbt.shape[1]
    nt = n // _TN
    return pl.pallas_call(
        _enc_vq_body,
        grid=(nt,),
        in_specs=[
            pl.BlockSpec((_TN, D), lambda i: (i, 0)),
            pl.BlockSpec((D, H), lambda i: (0, 0)),
            pl.BlockSpec((1, H), lambda i: (0, 0)),
            pl.BlockSpec((H, L), lambda i: (0, 0)),
            pl.BlockSpec((1, L), lambda i: (0, 0)),
            pl.BlockSpec((L, K), lambda i: (0, 0)),
            pl.BlockSpec((L, K), lambda i: (0, 0)),
        ],
        out_specs=[
            pl.BlockSpec((_TN, L), lambda i: (i, 0)),
            pl.BlockSpec((1, 1, _TN), lambda i: (i, 0, 0)),
        ],
        out_shape=[
            jax.ShapeDtypeStruct((n, L), jnp.float32),
            jax.ShapeDtypeStruct((nt, 1, _TN), jnp.int32),
        ],
        scratch_shapes=[pltpu.VMEM((1, K), jnp.float32)],
    )(xp, w1b, b1r, w2b, b2r, cbt, cbtb)


def _dec(q, z, w1b, b1r, w2b, b2r):
    n, L = q.shape
    H = w1b.shape[1]
    D = w2b.shape[1]
    nt = n // _TN
    return pl.pallas_call(
        _dec_body,
        grid=(nt,),
        in_specs=[
            pl.BlockSpec((_TN, L), lambda i: (i, 0)),
            pl.BlockSpec((_TN, L), lambda i: (i, 0)),
            pl.BlockSpec((L, H), lambda i: (0, 0)),
            pl.BlockSpec((1, H), lambda i: (0, 0)),
            pl.BlockSpec((H, D), lambda i: (0, 0)),
            pl.BlockSpec((1, D), lambda i: (0, 0)),
        ],
        out_specs=[
            pl.BlockSpec((_TN, D), lambda i: (i, 0)),
            pl.BlockSpec((1, 1), lambda i: (0, 0)),
        ],
        out_shape=[
            jax.ShapeDtypeStruct((n, D), jnp.float32),
            jax.ShapeDtypeStruct((1, 1), jnp.float32),
        ],
    )(q, z, w1b, b1r, w2b, b2r)


def kernel(x, enc_W1, enc_b1, enc_W2, enc_b2, codebook, dec_W1, dec_b1,
           dec_W2, dec_b2):
    N, D = x.shape
    H = enc_W1.shape[1]
    L = enc_W2.shape[1]

    cbt = codebook.T
    bf = jnp.bfloat16
    ew1, eb1 = enc_W1.astype(bf), enc_b1.reshape(1, H)
    ew2, eb2 = enc_W2.astype(bf), enc_b2.reshape(1, L)
    dw1, db1 = dec_W1.astype(bf), dec_b1.reshape(1, H)
    dw2, db2 = dec_W2.astype(bf), dec_b2.reshape(1, D)
    cbtb = cbt.astype(bf)

    z, i3 = _enc_vq(x, ew1, eb1, ew2, eb2, cbt, cbtb)
    idx = i3.reshape(N)
    q = _sc_gather(codebook, idx)
    x_recon, ls = _dec(q, z, dw1, db1, dw2, db2)

    vq_loss = 1.25 * (ls[0, 0] / (N * L))
    return (x_recon, vq_loss, idx[:, None])


# speedup vs baseline: 1.0418x; 1.0418x over previous
import functools

import jax
import jax.numpy as jnp
from jax import lax
from jax.experimental import pallas as pl
from jax.experimental.pallas import tpu as pltpu
from jax.experimental.pallas import tpu_sc as plsc

_TN = 512
_KC = 1024


def _dot(a, b):
    return lax.dot_general(a, b, (((1,), (0,)), ((), ())),
                           preferred_element_type=jnp.float32)


def _enc_vq_body(x_ref, w1_ref, b1_ref, w2_ref, b2_ref, cbt_ref, cbtb_ref,
                 z_ref, idx_ref, bsq_ref):
    K = cbt_ref.shape[1]

    @pl.when(pl.program_id(0) == 0)
    def _():
        cbt = cbt_ref[...]
        bsq_ref[...] = jnp.sum(cbt * cbt, axis=0, keepdims=True)

    h = jnp.maximum(_dot(x_ref[...].astype(jnp.bfloat16), w1_ref[...])
                    + b1_ref[...], 0.0)
    z = _dot(h.astype(jnp.bfloat16), w2_ref[...]) + b2_ref[...]
    a = jnp.sum(z * z, axis=1, keepdims=True)
    zb2 = (z * -2.0).astype(jnp.bfloat16)
    tn = z.shape[0]
    kc = min(_KC, K)
    run_min = None
    for c in range(0, K, kc):
        m2 = _dot(zb2, cbtb_ref[:, c:c + kc])
        d = (a + bsq_ref[:, c:c + kc]) + m2
        cmin = jnp.min(d, axis=1, keepdims=True)
        iota = lax.broadcasted_iota(jnp.int32, (1, kc), 1) + c
        cidx = jnp.min(jnp.where(d == cmin, iota, K), axis=1, keepdims=True)
        if run_min is None:
            run_min, run_idx = cmin, cidx
        else:
            better = cmin < run_min
            run_min = jnp.where(better, cmin, run_min)
            run_idx = jnp.where(better, cidx, run_idx)
    z_ref[...] = z
    idx_ref[0, 0, :] = run_idx.reshape(tn)


def _dec_body(q_ref, z_ref, w1_ref, b1_ref, w2_ref, b2_ref,
              xr_ref, loss_ref):
    q = q_ref[...].astype(jnp.bfloat16).astype(jnp.float32)
    z = z_ref[...]
    diff = q - z

    @pl.when(pl.program_id(0) == 0)
    def _():
        loss_ref[...] = jnp.zeros_like(loss_ref)

    loss_ref[...] += jnp.sum(diff * diff)
    qst = z + diff
    hd = jnp.maximum(_dot(qst.astype(jnp.bfloat16), w1_ref[...]) + b1_ref[...],
                     0.0)
    logits = _dot(hd.astype(jnp.bfloat16), w2_ref[...]) + b2_ref[...]
    xr_ref[...] = 1.0 / (1.0 + jnp.exp(-logits))


def _sc_gather(table, idx):
    n, d = idx.shape[0], table.shape[1]
    nw = 32
    bpw = n // nw
    mesh = plsc.VectorSubcoreMesh(core_axis_name="c", subcore_axis_name="s")

    @functools.partial(
        pl.kernel,
        mesh=mesh,
        out_type=jax.ShapeDtypeStruct((n, d), jnp.float32),
        scratch_types=[
            pltpu.VMEM((bpw,), jnp.int32),
            pltpu.VMEM((bpw, d), jnp.float32),
            pltpu.SemaphoreType.DMA,
        ],
    )
    def gather(table_hbm, idx_hbm, out_hbm, idx_v, rows_v, sem):
        wid = lax.axis_index("s") * 2 + lax.axis_index("c")
        base = wid * bpw
        pltpu.sync_copy(idx_hbm.at[pl.ds(base, bpw)], idx_v)
        pltpu.async_copy(table_hbm.at[idx_v], rows_v, sem).wait()
        pltpu.sync_copy(rows_v, out_hbm.at[pl.ds(base, bpw)])

    return gather(table, idx)


def _enc_vq(xp, w1b, b1r, w2b, b2r, cbt, cbtb):
    n, D = xp.shape
    H = w1b.shape[1]
    L = w2b.shape[1]
    K = cbt.shape[1]
    nt = n // _TN
    return pl.pallas_call(
        _enc_vq_body,
        grid=(nt,),
        in_specs=[
            pl.BlockSpec((_TN, D), lambda i: (i, 0)),
            pl.BlockSpec((D, H), lambda i: (0, 0)),
            pl.BlockSpec((1, H), lambda i: (0, 0)),
            pl.BlockSpec((H, L), lambda i: (0, 0)),
            pl.BlockSpec((1, L), lambda i: (0, 0)),
            pl.BlockSpec((L, K), lambda i: (0, 0)),
            pl.BlockSpec((L, K), lambda i: (0, 0)),
        ],
        out_specs=[
            pl.BlockSpec((_TN, L), lambda i: (i, 0)),
            pl.BlockSpec((1, 1, _TN), lambda i: (i, 0, 0)),
        ],
        out_shape=[
            jax.ShapeDtypeStruct((n, L), jnp.float32),
            jax.ShapeDtypeStruct((nt, 1, _TN), jnp.int32),
        ],
        scratch_shapes=[pltpu.VMEM((1, K), jnp.float32)],
    )(xp, w1b, b1r, w2b, b2r, cbt, cbtb)


def _dec(q, z, w1b, b1r, w2b, b2r):
    n, L = q.shape
    H = w1b.shape[1]
    D = w2b.shape[1]
    nt = n // _TN
    return pl.pallas_call(
        _dec_body,
        grid=(nt,),
        in_specs=[
            pl.BlockSpec((_TN, L), lambda i: (i, 0)),
            pl.BlockSpec((_TN, L), lambda i: (i, 0)),
            pl.BlockSpec((L, H), lambda i: (0, 0)),
            pl.BlockSpec((1, H), lambda i: (0, 0)),
            pl.BlockSpec((H, D), lambda i: (0, 0)),
            pl.BlockSpec((1, D), lambda i: (0, 0)),
        ],
        out_specs=[
            pl.BlockSpec((_TN, D), lambda i: (i, 0)),
            pl.BlockSpec((1, 1), lambda i: (0, 0)),
        ],
        out_shape=[
            jax.ShapeDtypeStruct((n, D), jnp.float32),
            jax.ShapeDtypeStruct((1, 1), jnp.float32),
        ],
    )(q, z, w1b, b1r, w2b, b2r)


def kernel(x, enc_W1, enc_b1, enc_W2, enc_b2, codebook, dec_W1, dec_b1,
           dec_W2, dec_b2):
    N, D = x.shape
    H = enc_W1.shape[1]
    L = enc_W2.shape[1]

    cbt = codebook.T
    bf = jnp.bfloat16
    ew1, eb1 = enc_W1.astype(bf), enc_b1.reshape(1, H)
    ew2, eb2 = enc_W2.astype(bf), enc_b2.reshape(1, L)
    dw1, db1 = dec_W1.astype(bf), dec_b1.reshape(1, H)
    dw2, db2 = dec_W2.astype(bf), dec_b2.reshape(1, D)
    cbtb = cbt.astype(bf)

    z, i3 = _enc_vq(x, ew1, eb1, ew2, eb2, cbt, cbtb)
    idx = i3.reshape(N)
    q = _sc_gather(codebook, idx)
    x_recon, ls = _dec(q, z, dw1, db1, dw2, db2)

    vq_loss = 1.25 * (ls[0, 0] / (N * L))
    return (x_recon, vq_loss, idx[:, None])
